# baseline (speedup 1.0000x reference)
import functools

import jax
import jax.numpy as jnp
from jax import lax
from jax.experimental import pallas as pl
from jax.experimental.pallas import tpu as pltpu
from jax.experimental.pallas import tpu_sc as plsc

N_TOK = 2048
D_IN = 4096
D_OUT = 4096
FAN_IN = 128

NC = 2
NS = 16
NW = NC * NS
LANES = 16

CHUNKS = 2
CHUNK_J = D_OUT // CHUNKS
JC = CHUNK_J // NW
NCOL = 8
SUBBLKS = JC // NCOL
KVECS = FAN_IN // LANES


def _densify_body(chunk_off, w_hbm, m_hbm, dense_hbm, blk0, blk1, wv, mv,
                  sem0, sem1):
    wid = lax.axis_index("s") * NC + lax.axis_index("c")
    j0 = chunk_off + wid * JC

    pltpu.sync_copy(w_hbm.at[pl.ds(j0, JC), :], wv)
    pltpu.sync_copy(m_hbm.at[pl.ds(j0, JC), :], mv)

    zeros = jnp.zeros((LANES,), jnp.float32)
    blks = (blk0, blk1)
    sems = (sem0, sem1)

    for blk in blks:
        def zero_row(c, carry, blk=blk):
            def zero_step(i, cc):
                off = i * (LANES * 8)
                for u in range(8):
                    blk[c, pl.ds(off + u * LANES, LANES)] = zeros
                return cc

            lax.fori_loop(0, D_IN // (LANES * 8), zero_step, 0)
            return carry

        lax.fori_loop(0, NCOL, zero_row, 0)

    def scat_step_fn(blk, r0, clear):
        def step(i, c):
            r = r0 + i // KVECS
            k0 = (i % KVECS) * LANES
            row = jnp.full((LANES,), i // KVECS, jnp.int32)
            idx = mv[r, pl.ds(k0, LANES)]
            if clear:
                plsc.store_scatter(blk, (row, idx), zeros)
            else:
                plsc.addupdate_scatter(blk, (row, idx), wv[r, pl.ds(k0, LANES)])
            return c

        lax.fori_loop(0, NCOL * KVECS, step, 0)

    def flush_copy(sb):
        b = sb % 2
        row0 = wid * JC + sb * NCOL
        return pltpu.make_async_copy(
            blks[b], dense_hbm.at[pl.ds(row0, NCOL), :], sems[b])

    for sb in range(SUBBLKS):
        b = sb % 2
        if sb >= 2:
            flush_copy(sb - 2).wait()
            scat_step_fn(blks[b], (sb - 2) * NCOL, clear=True)
        scat_step_fn(blks[b], sb * NCOL, clear=False)
        flush_copy(sb).start()

    flush_copy(SUBBLKS - 2).wait()
    flush_copy(SUBBLKS - 1).wait()


def _densify_chunk(w, m, chunk):
    mesh = plsc.VectorSubcoreMesh(core_axis_name="c", subcore_axis_name="s")
    return pl.kernel(
        functools.partial(_densify_body, chunk * CHUNK_J),
        out_type=jax.ShapeDtypeStruct((CHUNK_J, D_IN), jnp.float32),
        mesh=mesh,
        compiler_params=pltpu.CompilerParams(needs_layout_passes=False),
        scratch_types=[
            pltpu.VMEM((NCOL, D_IN), jnp.float32),
            pltpu.VMEM((NCOL, D_IN), jnp.float32),
            pltpu.VMEM((JC, FAN_IN), jnp.float32),
            pltpu.VMEM((JC, FAN_IN), jnp.int32),
            pltpu.SemaphoreType.DMA,
            pltpu.SemaphoreType.DMA,
        ],
        name=f"densify_chunk{chunk}",
    )(w, m)


M_BLK = 2048
N_BLK = 512
N_GRID = CHUNK_J // N_BLK


def _matmul_first_body(x_ref, w_ref, b_ref, o_ref):
    acc = lax.dot_general(
        x_ref[...],
        w_ref[...].astype(jnp.bfloat16),
        dimension_numbers=(((1,), (1,)), ((), ())),
        preferred_element_type=jnp.float32,
        precision=lax.Precision.DEFAULT,
    )
    o_ref[...] = acc + b_ref[...][None, :]


def _matmul_rest_body(x_ref, w_ref, b_ref, prev_ref, o_ref):
    del prev_ref
    _matmul_first_body(x_ref, w_ref, b_ref, o_ref)


def _matmul_chunk(x, wt_chunk, bias_chunk, chunk, prev_out=None):
    x_spec = pl.BlockSpec((M_BLK, D_IN), lambda n: (0, 0))
    w_spec = pl.BlockSpec((N_BLK, D_IN), lambda n: (n, 0))
    b_spec = pl.BlockSpec((N_BLK,), lambda n: (n,))
    col0 = chunk * N_GRID
    out_spec = pl.BlockSpec((M_BLK, N_BLK), lambda n: (0, n + col0))
    out_shape = jax.ShapeDtypeStruct((N_TOK, D_OUT), jnp.float32)
    if prev_out is None:
        return pl.pallas_call(
            _matmul_first_body,
            grid=(N_GRID,),
            in_specs=[x_spec, w_spec, b_spec],
            out_specs=out_spec,
            out_shape=out_shape,
        )(x, wt_chunk, bias_chunk)
    return pl.pallas_call(
        _matmul_rest_body,
        grid=(N_GRID,),
        in_specs=[x_spec, w_spec, b_spec,
                  pl.BlockSpec(memory_space=pl.ANY)],
        out_specs=out_spec,
        out_shape=out_shape,
        input_output_aliases={3: 0},
    )(x, wt_chunk, bias_chunk, prev_out)


@jax.jit
def kernel(input, condensed_weight, input_mask, bias):
    x16 = input.astype(jnp.bfloat16)

    out = None
    for chunk in range(CHUNKS):
        dense = _densify_chunk(condensed_weight, input_mask, chunk)
        bias_c = lax.slice(bias, (chunk * CHUNK_J,), ((chunk + 1) * CHUNK_J,))
        out = _matmul_chunk(x16, dense, bias_c, chunk, out)
    return out


# speedup vs baseline: 1.0162x; 1.0162x over previous
import functools

import jax
import jax.numpy as jnp
from jax import lax
from jax.experimental import pallas as pl
from jax.experimental.pallas import tpu as pltpu
from jax.experimental.pallas import tpu_sc as plsc

N_TOK = 2048
D_IN = 4096
D_OUT = 4096
FAN_IN = 128

NC = 2
NS = 16
NW = NC * NS
LANES = 16

CHUNKS = 2
CHUNK_J = D_OUT // CHUNKS
JC = CHUNK_J // NW
NCOL = 8
SUBBLKS = JC // NCOL
KVECS = FAN_IN // LANES


def _densify_body(chunk_off, w_hbm, m_hbm, dense_hbm, blk0, blk1, wv, mv,
                  sem0, sem1):
    wid = lax.axis_index("s") * NC + lax.axis_index("c")
    j0 = chunk_off + wid * JC

    pltpu.sync_copy(w_hbm.at[pl.ds(j0, JC), :], wv)
    pltpu.sync_copy(m_hbm.at[pl.ds(j0, JC), :], mv)

    zeros = jnp.zeros((LANES,), jnp.float32)
    blks = (blk0, blk1)
    sems = (sem0, sem1)

    for blk in blks:
        def zero_row(c, carry, blk=blk):
            def zero_step(i, cc):
                off = i * (LANES * 8)
                for u in range(8):
                    blk[c, pl.ds(off + u * LANES, LANES)] = zeros
                return cc

            lax.fori_loop(0, D_IN // (LANES * 8), zero_step, 0)
            return carry

        lax.fori_loop(0, NCOL, zero_row, 0)

    def scat_step_fn(blk, r0, clear):
        def step(i, c):
            r = r0 + i // KVECS
            k0 = (i % KVECS) * LANES
            row = jnp.full((LANES,), i // KVECS, jnp.int32)
            idx = mv[r, pl.ds(k0, LANES)]
            if clear:
                plsc.store_scatter(blk, (row, idx), zeros)
            else:
                plsc.addupdate_scatter(blk, (row, idx), wv[r, pl.ds(k0, LANES)])
            return c

        lax.fori_loop(0, NCOL * KVECS, step, 0)

    def flush_copy(sb):
        b = sb % 2
        row0 = wid * JC + sb * NCOL
        return pltpu.make_async_copy(
            blks[b], dense_hbm.at[pl.ds(row0, NCOL), :], sems[b])

    for sb in range(SUBBLKS):
        b = sb % 2
        if sb >= 2:
            flush_copy(sb - 2).wait()
            scat_step_fn(blks[b], (sb - 2) * NCOL, clear=True)
        scat_step_fn(blks[b], sb * NCOL, clear=False)
        flush_copy(sb).start()

    flush_copy(SUBBLKS - 2).wait()
    flush_copy(SUBBLKS - 1).wait()


def _densify_chunk(w, m, chunk):
    mesh = plsc.VectorSubcoreMesh(core_axis_name="c", subcore_axis_name="s")
    return pl.kernel(
        functools.partial(_densify_body, chunk * CHUNK_J),
        out_type=jax.ShapeDtypeStruct((CHUNK_J, D_IN), jnp.float32),
        mesh=mesh,
        compiler_params=pltpu.CompilerParams(needs_layout_passes=False),
        scratch_types=[
            pltpu.VMEM((NCOL, D_IN), jnp.float32),
            pltpu.VMEM((NCOL, D_IN), jnp.float32),
            pltpu.VMEM((JC, FAN_IN), jnp.float32),
            pltpu.VMEM((JC, FAN_IN), jnp.int32),
            pltpu.SemaphoreType.DMA,
            pltpu.SemaphoreType.DMA,
        ],
        name=f"densify_chunk{chunk}",
    )(w, m)


M_BLK = 2048
N_BLK = 256
N_GRID = CHUNK_J // N_BLK


def _matmul_first_body(x_ref, w_ref, b_ref, o_ref):
    acc = lax.dot_general(
        x_ref[...],
        w_ref[...].astype(jnp.bfloat16),
        dimension_numbers=(((1,), (1,)), ((), ())),
        preferred_element_type=jnp.float32,
        precision=lax.Precision.DEFAULT,
    )
    o_ref[...] = acc + b_ref[...][None, :]


def _matmul_rest_body(x_ref, w_ref, b_ref, prev_ref, o_ref):
    del prev_ref
    _matmul_first_body(x_ref, w_ref, b_ref, o_ref)


def _matmul_chunk(x, wt_chunk, bias_chunk, chunk, prev_out=None):
    x_spec = pl.BlockSpec((M_BLK, D_IN), lambda n: (0, 0))
    w_spec = pl.BlockSpec((N_BLK, D_IN), lambda n: (n, 0))
    b_spec = pl.BlockSpec((N_BLK,), lambda n: (n,))
    col0 = chunk * N_GRID
    out_spec = pl.BlockSpec((M_BLK, N_BLK), lambda n: (0, n + col0))
    out_shape = jax.ShapeDtypeStruct((N_TOK, D_OUT), jnp.float32)
    if prev_out is None:
        return pl.pallas_call(
            _matmul_first_body,
            grid=(N_GRID,),
            in_specs=[x_spec, w_spec, b_spec],
            out_specs=out_spec,
            out_shape=out_shape,
        )(x, wt_chunk, bias_chunk)
    return pl.pallas_call(
        _matmul_rest_body,
        grid=(N_GRID,),
        in_specs=[x_spec, w_spec, b_spec,
                  pl.BlockSpec(memory_space=pl.ANY)],
        out_specs=out_spec,
        out_shape=out_shape,
        input_output_aliases={3: 0},
    )(x, wt_chunk, bias_chunk, prev_out)


@jax.jit
def kernel(input, condensed_weight, input_mask, bias):
    x16 = input.astype(jnp.bfloat16)

    out = None
    for chunk in range(CHUNKS):
        dense = _densify_chunk(condensed_weight, input_mask, chunk)
        bias_c = lax.slice(bias, (chunk * CHUNK_J,), ((chunk + 1) * CHUNK_J,))
        out = _matmul_chunk(x16, dense, bias_c, chunk, out)
    return out


# speedup vs baseline: 1.0207x; 1.0044x over previous
import functools

import jax
import jax.numpy as jnp
from jax import lax
from jax.experimental import pallas as pl
from jax.experimental.pallas import tpu as pltpu
from jax.experimental.pallas import tpu_sc as plsc

N_TOK = 2048
D_IN = 4096
D_OUT = 4096
FAN_IN = 128

NC = 2
NS = 16
NW = NC * NS
LANES = 16

CHUNK_SIZES = (1536, 2560)
CHUNK_OFFS = (0, 1536)
NCOL = 8
KVECS = FAN_IN // LANES


def _densify_body(chunk_off, jc, w_hbm, m_hbm, dense_hbm, blk0, blk1, wv, mv,
                  sem0, sem1):
    subblks = jc // NCOL
    wid = lax.axis_index("s") * NC + lax.axis_index("c")
    j0 = chunk_off + wid * jc

    pltpu.sync_copy(w_hbm.at[pl.ds(j0, jc), :], wv)
    pltpu.sync_copy(m_hbm.at[pl.ds(j0, jc), :], mv)

    zeros = jnp.zeros((LANES,), jnp.float32)
    blks = (blk0, blk1)
    sems = (sem0, sem1)

    for blk in blks:
        def zero_row(c, carry, blk=blk):
            def zero_step(i, cc):
                off = i * (LANES * 8)
                for u in range(8):
                    blk[c, pl.ds(off + u * LANES, LANES)] = zeros
                return cc

            lax.fori_loop(0, D_IN // (LANES * 8), zero_step, 0)
            return carry

        lax.fori_loop(0, NCOL, zero_row, 0)

    def scat_step_fn(blk, r0, clear):
        def step(i, c):
            r = r0 + i // KVECS
            k0 = (i % KVECS) * LANES
            row = jnp.full((LANES,), i // KVECS, jnp.int32)
            idx = mv[r, pl.ds(k0, LANES)]
            if clear:
                plsc.store_scatter(blk, (row, idx), zeros)
            else:
                plsc.addupdate_scatter(blk, (row, idx), wv[r, pl.ds(k0, LANES)])
            return c

        lax.fori_loop(0, NCOL * KVECS, step, 0)

    def flush_copy(sb):
        b = sb % 2
        row0 = wid * jc + sb * NCOL
        return pltpu.make_async_copy(
            blks[b], dense_hbm.at[pl.ds(row0, NCOL), :], sems[b])

    for sb in range(subblks):
        b = sb % 2
        if sb >= 2:
            flush_copy(sb - 2).wait()
            scat_step_fn(blks[b], (sb - 2) * NCOL, clear=True)
        scat_step_fn(blks[b], sb * NCOL, clear=False)
        flush_copy(sb).start()

    flush_copy(subblks - 2).wait()
    flush_copy(subblks - 1).wait()


def _densify_chunk(w, m, chunk):
    chunk_j = CHUNK_SIZES[chunk]
    jc = chunk_j // NW
    mesh = plsc.VectorSubcoreMesh(core_axis_name="c", subcore_axis_name="s")
    return pl.kernel(
        functools.partial(_densify_body, CHUNK_OFFS[chunk], jc),
        out_type=jax.ShapeDtypeStruct((chunk_j, D_IN), jnp.float32),
        mesh=mesh,
        compiler_params=pltpu.CompilerParams(needs_layout_passes=False),
        scratch_types=[
            pltpu.VMEM((NCOL, D_IN), jnp.float32),
            pltpu.VMEM((NCOL, D_IN), jnp.float32),
            pltpu.VMEM((jc, FAN_IN), jnp.float32),
            pltpu.VMEM((jc, FAN_IN), jnp.int32),
            pltpu.SemaphoreType.DMA,
            pltpu.SemaphoreType.DMA,
        ],
        name=f"densify_chunk{chunk}",
    )(w, m)


M_BLK = 2048
N_BLK = 256


def _matmul_first_body(x_ref, w_ref, b_ref, o_ref):
    acc = lax.dot_general(
        x_ref[...],
        w_ref[...].astype(jnp.bfloat16),
        dimension_numbers=(((1,), (1,)), ((), ())),
        preferred_element_type=jnp.float32,
        precision=lax.Precision.DEFAULT,
    )
    o_ref[...] = acc + b_ref[...][None, :]


def _matmul_rest_body(x_ref, w_ref, b_ref, prev_ref, o_ref):
    del prev_ref
    _matmul_first_body(x_ref, w_ref, b_ref, o_ref)


def _matmul_chunk(x, wt_chunk, bias_chunk, chunk, prev_out=None):
    n_grid = CHUNK_SIZES[chunk] // N_BLK
    x_spec = pl.BlockSpec((M_BLK, D_IN), lambda n: (0, 0))
    w_spec = pl.BlockSpec((N_BLK, D_IN), lambda n: (n, 0))
    b_spec = pl.BlockSpec((N_BLK,), lambda n: (n,))
    col0 = CHUNK_OFFS[chunk] // N_BLK
    out_spec = pl.BlockSpec((M_BLK, N_BLK), lambda n: (0, n + col0))
    out_shape = jax.ShapeDtypeStruct((N_TOK, D_OUT), jnp.float32)
    if prev_out is None:
        return pl.pallas_call(
            _matmul_first_body,
            grid=(n_grid,),
            in_specs=[x_spec, w_spec, b_spec],
            out_specs=out_spec,
            out_shape=out_shape,
        )(x, wt_chunk, bias_chunk)
    return pl.pallas_call(
        _matmul_rest_body,
        grid=(n_grid,),
        in_specs=[x_spec, w_spec, b_spec,
                  pl.BlockSpec(memory_space=pl.ANY)],
        out_specs=out_spec,
        out_shape=out_shape,
        input_output_aliases={3: 0},
    )(x, wt_chunk, bias_chunk, prev_out)


@jax.jit
def kernel(input, condensed_weight, input_mask, bias):
    x16 = input.astype(jnp.bfloat16)

    out = None
    for chunk in range(len(CHUNK_SIZES)):
        dense = _densify_chunk(condensed_weight, input_mask, chunk)
        off = CHUNK_OFFS[chunk]
        bias_c = lax.slice(bias, (off,), (off + CHUNK_SIZES[chunk],))
        out = _matmul_chunk(x16, dense, bias_c, chunk, out)
    return out
